# baseline (device time: 12407 ns/iter reference)
import jax
import jax.numpy as jnp
from jax import lax
from jax.experimental import pallas as pl
from jax.experimental.pallas import tpu as pltpu


def kernel(x):
    m, n = x.shape
    mh = m // 2

    def body(x_ref, out_ref, send_sems, recv_sems):
        my_x = lax.axis_index("x")
        my_y = lax.axis_index("y")
        my_z = lax.axis_index("z")
        peer = (my_x, 1 - my_y, my_z)
        zp = my_z + 1 - 2 * (my_z % 2)
        diag = (my_x, 1 - my_y, zp)
        base = my_y * m

        barrier_sem = pltpu.get_barrier_semaphore()
        for nbr in (peer, diag):
            pl.semaphore_signal(
                barrier_sem, inc=1,
                device_id=nbr, device_id_type=pl.DeviceIdType.MESH,
            )
        pl.semaphore_wait(barrier_sem, 2)

        out_ref[pl.ds(base, mh), :] = x_ref[pl.ds(0, mh), :].astype(jnp.bfloat16)
        rdma_a = pltpu.make_async_remote_copy(
            src_ref=out_ref.at[pl.ds(base, mh), :],
            dst_ref=out_ref.at[pl.ds(base, mh), :],
            send_sem=send_sems.at[0],
            recv_sem=recv_sems.at[0],
            device_id=peer,
            device_id_type=pl.DeviceIdType.MESH,
        )
        rdma_a.start()

        out_ref[pl.ds(base + mh, mh), :] = (
            x_ref[pl.ds(mh, mh), :].astype(jnp.bfloat16)
        )
        rdma_b = pltpu.make_async_remote_copy(
            src_ref=out_ref.at[pl.ds(base + mh, mh), :],
            dst_ref=out_ref.at[pl.ds(base + mh, mh), :],
            send_sem=send_sems.at[1],
            recv_sem=recv_sems.at[1],
            device_id=diag,
            device_id_type=pl.DeviceIdType.MESH,
        )
        rdma_b.start()

        rdma_a.wait()
        rdma_b.wait()

    return pl.pallas_call(
        body,
        out_shape=jax.ShapeDtypeStruct((2 * m, n), jnp.bfloat16),
        in_specs=[pl.BlockSpec(memory_space=pltpu.VMEM)],
        out_specs=pl.BlockSpec(memory_space=pltpu.VMEM),
        scratch_shapes=[
            pltpu.SemaphoreType.DMA((2,)),
            pltpu.SemaphoreType.DMA((2,)),
        ],
        compiler_params=pltpu.CompilerParams(collective_id=0),
    )(x)


# device time: 11387 ns/iter; 1.0896x vs baseline; 1.0896x over previous
import jax
import jax.numpy as jnp
from jax import lax
from jax.experimental import pallas as pl
from jax.experimental.pallas import tpu as pltpu

N_CHUNKS = 2


def kernel(x):
    m, n = x.shape
    mc = m // N_CHUNKS

    def body(x_hbm_ref, out_ref, x_vmem, load_sem, send_sems, recv_sems):
        my_x = lax.axis_index("x")
        my_y = lax.axis_index("y")
        my_z = lax.axis_index("z")
        peer = (my_x, 1 - my_y, my_z)
        base = my_y * m

        barrier_sem = pltpu.get_barrier_semaphore()
        pl.semaphore_signal(
            barrier_sem, inc=1,
            device_id=peer, device_id_type=pl.DeviceIdType.MESH,
        )
        load = pltpu.make_async_copy(x_hbm_ref, x_vmem, load_sem)
        load.start()
        load.wait()

        out_ref[pl.ds(base, mc), :] = x_vmem[pl.ds(0, mc), :].astype(jnp.bfloat16)
        pl.semaphore_wait(barrier_sem, 1)

        rdmas = []
        for c in range(N_CHUNKS):
            if c > 0:
                out_ref[pl.ds(base + c * mc, mc), :] = (
                    x_vmem[pl.ds(c * mc, mc), :].astype(jnp.bfloat16)
                )
            rdma = pltpu.make_async_remote_copy(
                src_ref=out_ref.at[pl.ds(base + c * mc, mc), :],
                dst_ref=out_ref.at[pl.ds(base + c * mc, mc), :],
                send_sem=send_sems.at[c],
                recv_sem=recv_sems.at[c],
                device_id=peer,
                device_id_type=pl.DeviceIdType.MESH,
            )
            rdma.start()
            rdmas.append(rdma)
        for rdma in rdmas:
            rdma.wait()

    return pl.pallas_call(
        body,
        out_shape=jax.ShapeDtypeStruct((2 * m, n), jnp.bfloat16),
        in_specs=[pl.BlockSpec(memory_space=pl.ANY)],
        out_specs=pl.BlockSpec(memory_space=pltpu.VMEM),
        scratch_shapes=[
            pltpu.VMEM((m, n), x.dtype),
            pltpu.SemaphoreType.DMA,
            pltpu.SemaphoreType.DMA((N_CHUNKS,)),
            pltpu.SemaphoreType.DMA((N_CHUNKS,)),
        ],
        compiler_params=pltpu.CompilerParams(collective_id=0),
    )(x)
